# baseline (speedup 1.0000x reference)
import jax
import jax.numpy as jnp
from jax import lax
from jax.experimental import pallas as pl
from jax.experimental.pallas import tpu as pltpu
from jax.experimental.pallas import tpu_sc as plsc

NUM_EMB = 100000
DIM = 128
SEQS = 4096
SEQ_LEN = 50
BATCH = SEQS * SEQ_LEN
NUM_CORES = 2
NUM_SUBCORES = 16
NUM_WORKERS = NUM_CORES * NUM_SUBCORES
B_PER_W = BATCH // NUM_WORKERS
CHUNK = 400
_SIZES_EVEN = [CHUNK] * (B_PER_W // CHUNK)
_SIZES_ODD = [CHUNK // 2, CHUNK // 2] + [CHUNK] * (B_PER_W // CHUNK - 1)


def _emb_body(table_hbm, idx_hbm, out_hbm,
              idx_all, rows0, rows1, g0, g1, s0, s1):
    wid = lax.axis_index("s") * NUM_CORES + lax.axis_index("c")
    base = wid * B_PER_W

    rows = (rows0, rows1)
    gsem = (g0, g1)
    ssem = (s0, s1)

    pltpu.sync_copy(idx_hbm.at[pl.ds(base, B_PER_W)], idx_all)

    def pipeline(sizes):
        offs = [sum(sizes[:i]) for i in range(len(sizes))]
        n_chunks = len(sizes)

        def gather(i, b):
            n = sizes[i]
            dst = rows[b] if n == CHUNK else rows[b].at[pl.ds(0, n)]
            return pltpu.async_copy(
                table_hbm.at[idx_all.at[pl.ds(offs[i], n)]], dst, gsem[b])

        def scatter(i, b):
            n = sizes[i]
            src = rows[b] if n == CHUNK else rows[b].at[pl.ds(0, n)]
            return pltpu.async_copy(
                src, out_hbm.at[pl.ds(base + offs[i], n)], ssem[b])

        gath = gather(0, 0)
        scat = [None, None]
        for i in range(n_chunks):
            b = i % 2
            nb = 1 - b
            gath.wait()
            scat[b] = scatter(i, b)
            if i + 1 < n_chunks:
                if scat[nb] is not None:
                    scat[nb].wait()
                gath = gather(i + 1, nb)
        scat[0].wait()
        scat[1].wait()

    @pl.when(wid % 2 == 0)
    def _():
        pipeline(_SIZES_EVEN)

    @pl.when(wid % 2 == 1)
    def _():
        pipeline(_SIZES_ODD)


@jax.jit
def _embed(table, idx):
    mesh = plsc.VectorSubcoreMesh(core_axis_name="c", subcore_axis_name="s")
    return pl.kernel(
        _emb_body,
        mesh=mesh,
        out_type=jax.ShapeDtypeStruct((BATCH, DIM), jnp.float32),
        scratch_types=[
            pltpu.VMEM((B_PER_W,), jnp.int32),
            pltpu.VMEM((CHUNK, DIM), jnp.float32),
            pltpu.VMEM((CHUNK, DIM), jnp.float32),
            pltpu.SemaphoreType.DMA,
            pltpu.SemaphoreType.DMA,
            pltpu.SemaphoreType.DMA,
            pltpu.SemaphoreType.DMA,
        ],
    )(table, idx)


def kernel(token_ids, embedding_matrix):
    idx = token_ids.T.reshape(-1).astype(jnp.int32)
    out = _embed(embedding_matrix, idx)
    return out.reshape(SEQ_LEN, SEQS, DIM).swapaxes(0, 1)


# speedup vs baseline: 1.0258x; 1.0258x over previous
import jax
import jax.numpy as jnp
from jax import lax
from jax.experimental import pallas as pl
from jax.experimental.pallas import tpu as pltpu
from jax.experimental.pallas import tpu_sc as plsc

NUM_EMB = 100000
DIM = 128
SEQS = 4096
SEQ_LEN = 50
BATCH = SEQS * SEQ_LEN
NUM_CORES = 2
NUM_SUBCORES = 16
NUM_WORKERS = NUM_CORES * NUM_SUBCORES
B_PER_W = BATCH // NUM_WORKERS
CHUNK = 400
N_CHUNKS = B_PER_W // CHUNK


def _emb_body(table_hbm, idx_hbm, out_hbm,
              idx_all, rows0, rows1, g0, g1, s0, s1):
    wid = lax.axis_index("s") * NUM_CORES + lax.axis_index("c")
    base = wid * B_PER_W

    rows = (rows0, rows1)
    gsem = (g0, g1)
    ssem = (s0, s1)

    pltpu.sync_copy(idx_hbm.at[pl.ds(base, B_PER_W)], idx_all)

    def gather(i, b):
        return pltpu.async_copy(
            table_hbm.at[idx_all.at[pl.ds(i * CHUNK, CHUNK)]], rows[b], gsem[b])

    gath = gather(0, 0)
    scat = [None, None]
    for i in range(N_CHUNKS):
        b = i % 2
        nb = 1 - b
        gath.wait()
        scat[b] = pltpu.async_copy(
            rows[b], out_hbm.at[pl.ds(base + i * CHUNK, CHUNK)], ssem[b])
        if i + 1 < N_CHUNKS:
            if scat[nb] is not None:
                scat[nb].wait()
            gath = gather(i + 1, nb)
    scat[0].wait()
    scat[1].wait()


@jax.jit
def _embed(table, idx):
    mesh = plsc.VectorSubcoreMesh(core_axis_name="c", subcore_axis_name="s")
    return pl.kernel(
        _emb_body,
        mesh=mesh,
        out_type=jax.ShapeDtypeStruct((BATCH, DIM), jnp.float32),
        scratch_types=[
            pltpu.VMEM((B_PER_W,), jnp.int32),
            pltpu.VMEM((CHUNK, DIM), jnp.float32),
            pltpu.VMEM((CHUNK, DIM), jnp.float32),
            pltpu.SemaphoreType.DMA,
            pltpu.SemaphoreType.DMA,
            pltpu.SemaphoreType.DMA,
            pltpu.SemaphoreType.DMA,
        ],
    )(table, idx)


def kernel(token_ids, embedding_matrix):
    idx = token_ids.T.reshape(-1).astype(jnp.int32)
    out = _embed(embedding_matrix, idx)
    return out.reshape(SEQ_LEN, SEQS, DIM).swapaxes(0, 1)
